# Initial kernel scaffold; baseline (speedup 1.0000x reference)
import jax
import jax.numpy as jnp
from jax.experimental import pallas as pl


def kernel(input, targets):
    raise NotImplementedError("write your pallas kernel here")


# speedup vs baseline: 87.0229x; 87.0229x over previous
import functools

import jax
import jax.numpy as jnp
from jax import lax
from jax.experimental import pallas as pl
from jax.experimental.pallas import tpu as pltpu
from jax.experimental.pallas import tpu_sc as plsc

ZP = -10000000000.0
N, C, S = 32, 1000, 512
L = 50
P = 2 * L + 1
PP = 128


def _sc_gather(table, idx):
    info = plsc.get_sparse_core_info()
    nw = info.num_cores * info.num_subcores
    B = idx.shape[0]
    D = table.shape[1]
    b_per_w = B // nw

    mesh = plsc.VectorSubcoreMesh(core_axis_name="c", subcore_axis_name="s")

    @functools.partial(
        pl.kernel,
        mesh=mesh,
        out_type=jax.ShapeDtypeStruct((B, D), jnp.float32),
        scratch_types=[
            pltpu.VMEM((b_per_w,), jnp.int32),
            pltpu.VMEM((b_per_w, D), jnp.float32),
            pltpu.SemaphoreType.DMA,
        ],
    )
    def k(table_hbm, idx_hbm, out_hbm, idx_v, rows_v, sem):
        wid = lax.axis_index("s") * info.num_cores + lax.axis_index("c")
        base = wid * b_per_w
        pltpu.sync_copy(idx_hbm.at[pl.ds(base, b_per_w)], idx_v)
        pltpu.async_copy(table_hbm.at[idx_v], rows_v, sem).wait()
        pltpu.sync_copy(rows_v, out_hbm.at[pl.ds(base, b_per_w)])

    return k(table, idx)


def _lse_body(x_ref, out_ref):
    x = x_ref[0]
    m = jnp.max(x, axis=0)
    s = jnp.sum(jnp.exp(x - m[None, :]), axis=0)
    out_ref[0, 0] = m + jnp.log(s)


def _transpose_body(rows_ref, lse_ref, g_ref):
    for i in range(rows_ref.shape[0]):
        x = rows_ref[i]
        g_ref[:, i, :] = x.T - lse_ref[i, 0][:, None]


def _rec_body(g_ref, same_ref, out_ref):
    same = same_ref[...] > 0.0
    lane = lax.broadcasted_iota(jnp.int32, (N, PP), 1)
    init = jnp.where(lane == 0, 0.0, ZP).astype(jnp.float32)

    def step(s, alpha):
        m1 = jnp.where(lane >= 1, jnp.roll(alpha, 1, axis=1), ZP)
        m2 = jnp.where((lane >= 2) & ~same, jnp.roll(alpha, 2, axis=1), ZP)
        vmax = jnp.maximum(alpha, jnp.maximum(m1, m2))
        t = vmax + jnp.log(
            jnp.exp(alpha - vmax) + jnp.exp(m1 - vmax) + jnp.exp(m2 - vmax))
        return t + g_ref[s]

    alpha = lax.fori_loop(0, S, step, init)
    sel = jnp.where((lane == P - 2) | (lane == P - 1), alpha, ZP)
    m = jnp.max(sel, axis=1, keepdims=True)
    loss = -(m + jnp.log(jnp.sum(jnp.exp(sel - m), axis=1, keepdims=True)))
    out_ref[...] = jnp.broadcast_to(loss, (N, PP))


def kernel(input, targets):
    path = jnp.zeros((N, PP), jnp.int32).at[:, 1:P:2].set(targets.astype(jnp.int32))
    idx = (jnp.arange(N, dtype=jnp.int32)[:, None] * C + path).reshape(N * PP)
    same = jnp.concatenate(
        [jnp.zeros((N, 2), jnp.float32),
         (path[:, :P - 2] == path[:, 2:P]).astype(jnp.float32),
         jnp.zeros((N, PP - P), jnp.float32)], axis=1)

    lse = pl.pallas_call(
        _lse_body,
        grid=(N,),
        in_specs=[pl.BlockSpec((1, C, S), lambda n: (n, 0, 0))],
        out_specs=pl.BlockSpec((1, 1, S), lambda n: (n, 0, 0)),
        out_shape=jax.ShapeDtypeStruct((N, 1, S), jnp.float32),
    )(input)

    rows = _sc_gather(input.reshape(N * C, S), idx)

    g = pl.pallas_call(
        _transpose_body,
        grid=(N // 8,),
        in_specs=[
            pl.BlockSpec((8, PP, S), lambda n: (n, 0, 0)),
            pl.BlockSpec((8, 1, S), lambda n: (n, 0, 0)),
        ],
        out_specs=pl.BlockSpec((S, 8, PP), lambda n: (0, n, 0)),
        out_shape=jax.ShapeDtypeStruct((S, N, PP), jnp.float32),
    )(rows.reshape(N, PP, S), lse)

    out = pl.pallas_call(
        _rec_body,
        in_specs=[
            pl.BlockSpec((S, N, PP), lambda: (0, 0, 0)),
            pl.BlockSpec((N, PP), lambda: (0, 0)),
        ],
        out_specs=pl.BlockSpec((N, PP), lambda: (0, 0)),
        out_shape=jax.ShapeDtypeStruct((N, PP), jnp.float32),
        grid=(),
    )(g, same)
    return out[:, 0]


# speedup vs baseline: 104.7171x; 1.2033x over previous
import functools

import jax
import jax.numpy as jnp
from jax import lax
from jax.experimental import pallas as pl
from jax.experimental.pallas import tpu as pltpu
from jax.experimental.pallas import tpu_sc as plsc

ZP = -10000000000.0
N, C, S = 32, 1000, 512
L = 50
P = 2 * L + 1
PP = 128


def _sc_gather(table, idx):
    info = plsc.get_sparse_core_info()
    nw = info.num_cores * info.num_subcores
    B = idx.shape[0]
    D = table.shape[1]
    b_per_w = B // nw

    mesh = plsc.VectorSubcoreMesh(core_axis_name="c", subcore_axis_name="s")

    @functools.partial(
        pl.kernel,
        mesh=mesh,
        out_type=jax.ShapeDtypeStruct((B, D), jnp.float32),
        scratch_types=[
            pltpu.VMEM((b_per_w,), jnp.int32),
            pltpu.VMEM((b_per_w, D), jnp.float32),
            pltpu.SemaphoreType.DMA,
        ],
    )
    def k(table_hbm, idx_hbm, out_hbm, idx_v, rows_v, sem):
        wid = lax.axis_index("s") * info.num_cores + lax.axis_index("c")
        base = wid * b_per_w
        pltpu.sync_copy(idx_hbm.at[pl.ds(base, b_per_w)], idx_v)
        pltpu.async_copy(table_hbm.at[idx_v], rows_v, sem).wait()
        pltpu.sync_copy(rows_v, out_hbm.at[pl.ds(base, b_per_w)])

    return k(table, idx)


def _lse_body(x_ref, out_ref):
    x = x_ref[0]
    m = jnp.max(x, axis=0)
    s = jnp.sum(jnp.exp(x - m[None, :]), axis=0)
    out_ref[0, 0] = m + jnp.log(s)


def _lse3(a, b, c):
    vmax = jnp.maximum(a, jnp.maximum(b, c))
    return vmax + jnp.log(
        jnp.exp(a - vmax) + jnp.exp(b - vmax) + jnp.exp(c - vmax))


def _rec_body(rows_ref, lse_ref, same_ref, sameb_ref, out_ref, g_ref):
    for n in range(N):
        g_ref[:, n, :] = rows_ref[n].T - lse_ref[n, 0][:, None]

    same = same_ref[...] > 0.0
    sameb = sameb_ref[...] > 0.0
    lane = lax.broadcasted_iota(jnp.int32, (N, PP), 1)
    f32 = jnp.float32
    initA = jnp.where(lane == 0, 0.0, ZP).astype(f32)
    initD = jnp.full((N, PP), ZP, f32)
    endI = jnp.where((lane == P - 1) | (lane == P - 2), 0.0, ZP).astype(f32)

    def transf(A):
        m1 = jnp.where(lane >= 1, jnp.roll(A, 1, axis=1), ZP)
        m2 = jnp.where((lane >= 2) & ~same, jnp.roll(A, 2, axis=1), ZP)
        return _lse3(A, m1, m2)

    def transb(D):
        m1 = jnp.where(lane <= P - 2, jnp.roll(D, -1, axis=1), ZP)
        m2 = jnp.where((lane <= P - 3) & ~sameb, jnp.roll(D, -2, axis=1), ZP)
        return _lse3(D, m1, m2)

    def step(i, carry):
        A, D = carry
        A = transf(A) + g_ref[i]
        Dn = transb(D) + g_ref[S - 1 - i]
        D0 = endI + g_ref[S - 1]
        D = jnp.where(i == 0, D0, Dn)
        return A, D

    A, D = lax.fori_loop(0, S // 2, step, (initA, initD))
    B = transb(D)
    sel = jnp.where(lane <= P - 1, A + B, ZP)
    m = jnp.max(sel, axis=1, keepdims=True)
    loss = -(m + jnp.log(jnp.sum(jnp.exp(sel - m), axis=1, keepdims=True)))
    out_ref[...] = jnp.broadcast_to(loss, (N, PP))


def kernel(input, targets):
    path = jnp.zeros((N, PP), jnp.int32).at[:, 1:P:2].set(targets.astype(jnp.int32))
    idx = (jnp.arange(N, dtype=jnp.int32)[:, None] * C + path).reshape(N * PP)
    same_b = jnp.concatenate(
        [jnp.zeros((N, 2), jnp.bool_),
         path[:, :P - 2] == path[:, 2:P],
         jnp.zeros((N, PP - P), jnp.bool_)], axis=1)
    sameb_b = jnp.concatenate([same_b[:, 2:], jnp.ones((N, 2), jnp.bool_)], axis=1)
    same = same_b.astype(jnp.float32)
    sameb = sameb_b.astype(jnp.float32)

    lse = pl.pallas_call(
        _lse_body,
        grid=(N,),
        in_specs=[pl.BlockSpec((1, C, S), lambda n: (n, 0, 0))],
        out_specs=pl.BlockSpec((1, 1, S), lambda n: (n, 0, 0)),
        out_shape=jax.ShapeDtypeStruct((N, 1, S), jnp.float32),
    )(input)

    rows = _sc_gather(input.reshape(N * C, S), idx)

    out = pl.pallas_call(
        _rec_body,
        in_specs=[
            pl.BlockSpec((N, PP, S), lambda: (0, 0, 0)),
            pl.BlockSpec((N, 1, S), lambda: (0, 0, 0)),
            pl.BlockSpec((N, PP), lambda: (0, 0)),
            pl.BlockSpec((N, PP), lambda: (0, 0)),
        ],
        out_specs=pl.BlockSpec((N, PP), lambda: (0, 0)),
        out_shape=jax.ShapeDtypeStruct((N, PP), jnp.float32),
        scratch_shapes=[pltpu.VMEM((S, N, PP), jnp.float32)],
        grid=(),
    )(rows.reshape(N, PP, S), lse, same, sameb)
    return out[:, 0]
